# baseline (speedup 1.0000x reference)
import jax
import jax.numpy as jnp
from jax import lax
from jax.experimental import pallas as pl
from jax.experimental.pallas import tpu as pltpu
from jax.experimental.pallas import tpu_sc as plsc

N = 10000
E = 160000
H = 256
NL = 5
G = 64
C = 10

NC = 2
NS = 16
HN = H // 2
EPT = E // NS
K = 80
NCHUNK = EPT // K
RPT = 624
RPT_LAST = N - (NS - 1) * RPT

BN_ = 2000
NB = N // BN_


def _seg_sum_body(h2, src4, dst3, zer, out, sidx, didx, rows0, rows1, acc,
                  gsem0, gsem1, ssem0, ssem1):
    c = lax.axis_index("c")
    s = lax.axis_index("s")
    pltpu.sync_copy(src4.at[c, s], sidx)
    pltpu.sync_copy(dst3.at[s], didx)

    def sx(j):
        return sidx.at[pl.ds(j * K, K)]

    @pl.when(s < NS - 1)
    def _():
        pltpu.sync_copy(zer.at[pl.ds(0, RPT)], acc.at[pl.ds(s * RPT, RPT)])

    @pl.when(s == NS - 1)
    def _():
        pltpu.sync_copy(zer, acc.at[pl.ds((NS - 1) * RPT, RPT_LAST)])

    plsc.subcore_barrier()

    pltpu.async_copy(h2.at[sx(0)], rows0, gsem0)
    pltpu.async_copy(h2.at[sx(1)], rows1, gsem1)

    def body(jj, carry):
        j0 = 2 * jj
        pltpu.make_async_copy(h2.at[sx(j0)], rows0, gsem0).wait()
        pltpu.async_copy(rows0, acc.at[didx.at[j0]], ssem0, add=True)
        pltpu.make_async_copy(h2.at[sx(j0 + 1)], rows1, gsem1).wait()
        pltpu.async_copy(rows1, acc.at[didx.at[j0 + 1]], ssem1, add=True)
        pltpu.make_async_copy(rows0, acc.at[didx.at[j0]], ssem0).wait()
        pltpu.async_copy(h2.at[sx(j0 + 2)], rows0, gsem0)
        pltpu.make_async_copy(rows1, acc.at[didx.at[j0 + 1]], ssem1).wait()
        pltpu.async_copy(h2.at[sx(j0 + 3)], rows1, gsem1)
        return carry

    lax.fori_loop(0, (NCHUNK - 3) // 2, body, 0)

    jt = 2 * ((NCHUNK - 3) // 2)
    pltpu.make_async_copy(h2.at[sx(jt)], rows0, gsem0).wait()
    pltpu.async_copy(rows0, acc.at[didx.at[jt]], ssem0, add=True)
    pltpu.make_async_copy(h2.at[sx(jt + 1)], rows1, gsem1).wait()
    pltpu.async_copy(rows1, acc.at[didx.at[jt + 1]], ssem1, add=True)
    pltpu.make_async_copy(rows0, acc.at[didx.at[jt]], ssem0).wait()
    pltpu.make_async_copy(rows1, acc.at[didx.at[jt + 1]], ssem1).wait()
    jl = NCHUNK - 1
    pltpu.async_copy(h2.at[sx(jl)], rows0, gsem0).wait()
    pltpu.async_copy(rows0, acc.at[didx.at[jl]], ssem0, add=True)
    pltpu.make_async_copy(rows0, acc.at[didx.at[jl]], ssem0).wait()

    plsc.subcore_barrier()

    @pl.when(s < NS - 1)
    def _():
        pltpu.sync_copy(acc.at[pl.ds(s * RPT, RPT)],
                        out.at[pl.ds(c * N + s * RPT, RPT)])

    @pl.when(s == NS - 1)
    def _():
        pltpu.sync_copy(acc.at[pl.ds((NS - 1) * RPT, RPT_LAST)],
                        out.at[pl.ds(c * N + (NS - 1) * RPT, RPT_LAST)])


_seg_sum_cache = None


def _get_seg_sum():
    global _seg_sum_cache
    if _seg_sum_cache is None:
        _seg_sum_cache = pl.kernel(
            _seg_sum_body,
            out_type=jax.ShapeDtypeStruct((2 * N, HN), jnp.float32),
            mesh=plsc.VectorSubcoreMesh(core_axis_name="c",
                                        subcore_axis_name="s",
                                        num_cores=NC, num_subcores=NS),
            scratch_types=[
                pltpu.VMEM((EPT,), jnp.int32),
                pltpu.VMEM((NCHUNK, K), jnp.int32),
                pltpu.VMEM((K, HN), jnp.float32),
                pltpu.VMEM((K, HN), jnp.float32),
                pltpu.VMEM_SHARED((N, HN), jnp.float32),
                pltpu.SemaphoreType.DMA,
                pltpu.SemaphoreType.DMA,
                pltpu.SemaphoreType.DMA,
                pltpu.SemaphoreType.DMA,
            ],
        )
    return _seg_sum_cache


def _mm(a, b):
    return lax.dot_general(a, b, (((1,), (0,)), ((), ())),
                           preferred_element_type=jnp.float32)


def _layer_compute(h0, h1, a0, a1, Wa, ba, Wb, bb, zbuf, stats, i):
    u = jnp.concatenate([h0[...] + a0[...], h1[...] + a1[...]], axis=1)
    z = jnp.maximum(_mm(u, Wa[...]) + ba[...], 0.0)
    z = jnp.maximum(_mm(z, Wb[...]) + bb[...], 0.0)
    zbuf[pl.ds(i * BN_, BN_), :] = z
    stats[0:1, :] += jnp.sum(z, axis=0, keepdims=True)
    stats[1:2, :] += jnp.sum(z * z, axis=0, keepdims=True)


def _bn_coeffs(stats, g, b):
    mu = stats[0:1, :] * (1.0 / N)
    var = stats[1:2, :] * (1.0 / N) - mu * mu
    scale = g[...] * lax.rsqrt(var + 1e-5)
    shift = b[...] - mu * scale
    return scale, shift


def _mlp_mid_body(h0, h1, a0, a1, Wa, ba, Wb, bb, g, b, out, zbuf, stats):
    i = pl.program_id(0)

    @pl.when(i == 0)
    def _():
        stats[...] = jnp.zeros_like(stats)

    @pl.when(i < NB)
    def _():
        _layer_compute(h0, h1, a0, a1, Wa, ba, Wb, bb, zbuf, stats, i)

    @pl.when((i >= NB) & (i < 2 * NB))
    def _():
        scale, shift = _bn_coeffs(stats, g, b)
        zn = zbuf[pl.ds((i - NB) * BN_, BN_), :] * scale + shift
        out[...] = zn[:, :HN]

    @pl.when(i >= 2 * NB)
    def _():
        scale, shift = _bn_coeffs(stats, g, b)
        zn = zbuf[pl.ds((i - 2 * NB) * BN_, BN_), :] * scale + shift
        out[...] = zn[:, HN:]


def _mlp_last_body(h0, h1, a0, a1, Wa, ba, Wb, bb, g, b, out, zbuf, stats):
    i = pl.program_id(0)

    @pl.when(i == 0)
    def _():
        stats[...] = jnp.zeros_like(stats)

    @pl.when(i < NB)
    def _():
        _layer_compute(h0, h1, a0, a1, Wa, ba, Wb, bb, zbuf, stats, i)

    @pl.when(i >= NB)
    def _():
        scale, shift = _bn_coeffs(stats, g, b)
        out[...] = zbuf[pl.ds((i - NB) * BN_, BN_), :] * scale + shift


def _clampi(i):
    return jnp.minimum(i, NB - 1)


_COMMON_IN_SPECS = [
    pl.BlockSpec((BN_, HN), lambda i: (_clampi(i), 0)),
    pl.BlockSpec((BN_, HN), lambda i: (NB + _clampi(i), 0)),
    pl.BlockSpec((BN_, HN), lambda i: (_clampi(i), 0)),
    pl.BlockSpec((BN_, HN), lambda i: (NB + _clampi(i), 0)),
    pl.BlockSpec((H, H), lambda i: (0, 0)),
    pl.BlockSpec((1, H), lambda i: (0, 0)),
    pl.BlockSpec((H, H), lambda i: (0, 0)),
    pl.BlockSpec((1, H), lambda i: (0, 0)),
    pl.BlockSpec((1, H), lambda i: (0, 0)),
    pl.BlockSpec((1, H), lambda i: (0, 0)),
]

_mlp_mid = pl.pallas_call(
    _mlp_mid_body,
    grid=(3 * NB,),
    in_specs=_COMMON_IN_SPECS,
    out_specs=pl.BlockSpec((BN_, HN), lambda i: (jnp.maximum(i - NB, 0), 0)),
    out_shape=jax.ShapeDtypeStruct((2 * N, HN), jnp.float32),
    scratch_shapes=[pltpu.VMEM((N, H), jnp.float32),
                    pltpu.VMEM((8, H), jnp.float32)],
    compiler_params=pltpu.CompilerParams(
        dimension_semantics=("arbitrary",)),
)

_mlp_last = pl.pallas_call(
    _mlp_last_body,
    grid=(2 * NB,),
    in_specs=_COMMON_IN_SPECS,
    out_specs=pl.BlockSpec((BN_, H), lambda i: (jnp.maximum(i - NB, 0), 0)),
    out_shape=jax.ShapeDtypeStruct((N, H), jnp.float32),
    scratch_shapes=[pltpu.VMEM((N, H), jnp.float32),
                    pltpu.VMEM((8, H), jnp.float32)],
    compiler_params=pltpu.CompilerParams(
        dimension_semantics=("arbitrary",)),
)


def _pool_head_body(hf, batch, W1, b1, W2, b2, out, pooled):
    i = pl.program_id(0)

    @pl.when(i == 0)
    def _():
        pooled[...] = jnp.zeros_like(pooled)

    oh = (batch[...] == lax.broadcasted_iota(jnp.int32, (BN_, G), 1))
    oh = oh.astype(jnp.float32)
    zaug = jnp.concatenate(
        [hf[...], jnp.ones((BN_, HN), jnp.float32)], axis=1)
    pooled[...] += lax.dot_general(oh, zaug, (((0,), (0,)), ((), ())),
                                   preferred_element_type=jnp.float32)

    @pl.when(i == NB - 1)
    def _():
        P = pooled[...]
        cnt = P[:, H:H + 1]
        pm = P[:, :H] / jnp.maximum(cnt, 1.0)
        o = jnp.maximum(_mm(pm, W1[...]) + b1[...], 0.0)
        o = _mm(o, W2[...]) + b2[...]
        m = jnp.max(o, axis=1, keepdims=True)
        lse = jnp.log(jnp.sum(jnp.exp(o - m), axis=1, keepdims=True))
        out[...] = o - m - lse


_pool_head = pl.pallas_call(
    _pool_head_body,
    grid=(NB,),
    in_specs=[
        pl.BlockSpec((BN_, H), lambda i: (i, 0)),
        pl.BlockSpec((BN_, 1), lambda i: (i, 0)),
        pl.BlockSpec((H, H), lambda i: (0, 0)),
        pl.BlockSpec((1, H), lambda i: (0, 0)),
        pl.BlockSpec((H, C), lambda i: (0, 0)),
        pl.BlockSpec((1, C), lambda i: (0, 0)),
    ],
    out_specs=pl.BlockSpec((G, C), lambda i: (0, 0)),
    out_shape=jax.ShapeDtypeStruct((G, C), jnp.float32),
    scratch_shapes=[pltpu.VMEM((G, H + HN), jnp.float32)],
    compiler_params=pltpu.CompilerParams(
        dimension_semantics=("arbitrary",)),
)


def kernel(x, edge_index, batch, Wa, ba, Wb, bb, gamma, beta, W1, b1, W2, b2):
    src = edge_index[0].astype(jnp.int32)
    dst = edge_index[1].astype(jnp.int32)
    src4 = jnp.stack([src, src + N]).reshape(NC, NS, EPT)
    dst3 = dst.reshape(NS, NCHUNK, K)
    zer = jnp.zeros((RPT_LAST, HN), jnp.float32)
    batch2 = batch.reshape(N, 1).astype(jnp.int32)
    h2 = x.reshape(N, 2, HN).transpose(1, 0, 2).reshape(2 * N, HN)
    seg_sum = _get_seg_sum()
    for l in range(NL):
        agg2 = seg_sum(h2, src4, dst3, zer)
        args = (h2, h2, agg2, agg2, Wa[l], ba[l].reshape(1, H), Wb[l],
                bb[l].reshape(1, H), gamma[l].reshape(1, H),
                beta[l].reshape(1, H))
        if l < NL - 1:
            h2 = _mlp_mid(*args)
        else:
            hf = _mlp_last(*args)
    return _pool_head(hf, batch2, W1, b1.reshape(1, H), W2,
                      b2.reshape(1, C))


# speedup vs baseline: 1.4713x; 1.4713x over previous
import jax
import jax.numpy as jnp
from jax import lax
from jax.experimental import pallas as pl
from jax.experimental.pallas import tpu as pltpu
from jax.experimental.pallas import tpu_sc as plsc

N = 10000
E = 160000
H = 256
NL = 5
G = 64
C = 10

NC = 2
NS = 16
HN = H // 2
EPT = E // NS
K = 80
NCHUNK = EPT // K
RPT = 624
RPT_LAST = N - (NS - 1) * RPT

BN_ = 2000
NB = N // BN_


def _seg_sum_body(h2, src2, dst2, zer, out,
                  si0, si1, si2, si3, si4, si5, dd0, dd1, dd2,
                  rw0, rw1, rw2, acc,
                  is0, is1, is2, is3, is4, is5, dsm0, dsm1, dsm2,
                  gs0, gs1, gs2, ss0, ss1, ss2):
    sis = [si0, si1, si2, si3, si4, si5]
    dds = [dd0, dd1, dd2]
    rws = [rw0, rw1, rw2]
    isem = [is0, is1, is2, is3, is4, is5]
    dsem = [dsm0, dsm1, dsm2]
    gsem = [gs0, gs1, gs2]
    ssem = [ss0, ss1, ss2]
    c = lax.axis_index("c")
    s = lax.axis_index("s")
    ebase = c * E + s * EPT
    dbase = s * EPT

    def iload(j, q):
        pltpu.async_copy(src2.at[pl.ds(ebase + j * K, K)], sis[q], isem[q])

    def dload(j, b):
        pltpu.async_copy(dst2.at[pl.ds(dbase + j * K, K)], dds[b], dsem[b])

    def gather(q, b):
        pltpu.async_copy(h2.at[sis[q]], rws[b], gsem[b])

    def wait_iload(q):
        pltpu.make_async_copy(src2.at[pl.ds(dbase, K)], sis[q],
                              isem[q]).wait()

    def wait_dload(b):
        pltpu.make_async_copy(dst2.at[pl.ds(dbase, K)], dds[b],
                              dsem[b]).wait()

    def wait_gather(b):
        pltpu.make_async_copy(h2.at[sis[b]], rws[b], gsem[b]).wait()

    def wait_scatter(b):
        pltpu.make_async_copy(rws[b], acc.at[dds[b]], ssem[b]).wait()

    @pl.when(s < NS - 1)
    def _():
        pltpu.sync_copy(zer.at[pl.ds(0, RPT)], acc.at[pl.ds(s * RPT, RPT)])

    @pl.when(s == NS - 1)
    def _():
        pltpu.sync_copy(zer, acc.at[pl.ds((NS - 1) * RPT, RPT_LAST)])

    plsc.subcore_barrier()

    for q in range(6):
        iload(q, q)
    for b in range(3):
        dload(b, b)
    for b in range(3):
        wait_iload(b)
        gather(b, b)

    GRP = NCHUNK // 6
    TAIL = GRP * 6

    def emit(j, u, in_loop):
        b, q = u % 3, u
        wait_gather(b)
        wait_dload(b)
        pltpu.async_copy(rws[b], acc.at[dds[b]], ssem[b], add=True)
        nxt_load = j + 6 < NCHUNK
        nxt_gather = j + 3 < NCHUNK

        def advance():
            wait_scatter(b)
            dload(j + 3, b)
            wait_iload((u + 3) % 6)
            gather((u + 3) % 6, b)

        if in_loop:
            @pl.when(nxt_load)
            def _():
                iload(j + 6, q)

            @pl.when(nxt_gather)
            def _():
                advance()
        else:
            if nxt_load:
                iload(j + 6, q)
            if nxt_gather:
                advance()

    def body(g, carry):
        for u in range(6):
            emit(6 * g + u, u, True)
        return carry

    lax.fori_loop(0, GRP, body, 0)
    for j in range(TAIL, NCHUNK):
        emit(j, j % 6, False)
    for b in range(3):
        wait_scatter(b)

    plsc.subcore_barrier()

    @pl.when(s < NS - 1)
    def _():
        pltpu.sync_copy(acc.at[pl.ds(s * RPT, RPT)],
                        out.at[pl.ds(c * N + s * RPT, RPT)])

    @pl.when(s == NS - 1)
    def _():
        pltpu.sync_copy(acc.at[pl.ds((NS - 1) * RPT, RPT_LAST)],
                        out.at[pl.ds(c * N + (NS - 1) * RPT, RPT_LAST)])


_seg_sum_cache = None


def _get_seg_sum():
    global _seg_sum_cache
    if _seg_sum_cache is None:
        _seg_sum_cache = pl.kernel(
            _seg_sum_body,
            out_type=jax.ShapeDtypeStruct((2 * N, HN), jnp.float32),
            mesh=plsc.VectorSubcoreMesh(core_axis_name="c",
                                        subcore_axis_name="s",
                                        num_cores=NC, num_subcores=NS),
            scratch_types=(
                [pltpu.VMEM((K,), jnp.int32) for _ in range(6)]
                + [pltpu.VMEM((K,), jnp.int32) for _ in range(3)]
                + [pltpu.VMEM((K, HN), jnp.float32) for _ in range(3)]
                + [pltpu.VMEM_SHARED((N, HN), jnp.float32)]
                + [pltpu.SemaphoreType.DMA for _ in range(15)]
            ),
        )
    return _seg_sum_cache


def _mm(a, b):
    return lax.dot_general(a, b, (((1,), (0,)), ((), ())),
                           preferred_element_type=jnp.float32)


def _layer_compute(h0, h1, a0, a1, Wa, ba, Wb, bb, zbuf, stats, i):
    u = jnp.concatenate([h0[...] + a0[...], h1[...] + a1[...]], axis=1)
    z = jnp.maximum(_mm(u, Wa[...]) + ba[...], 0.0)
    z = jnp.maximum(_mm(z, Wb[...]) + bb[...], 0.0)
    zbuf[pl.ds(i * BN_, BN_), :] = z
    stats[0:1, :] += jnp.sum(z, axis=0, keepdims=True)
    stats[1:2, :] += jnp.sum(z * z, axis=0, keepdims=True)


def _bn_coeffs(stats, g, b):
    mu = stats[0:1, :] * (1.0 / N)
    var = stats[1:2, :] * (1.0 / N) - mu * mu
    scale = g[...] * lax.rsqrt(var + 1e-5)
    shift = b[...] - mu * scale
    return scale, shift


def _mlp_mid_body(h0, h1, a0, a1, Wa, ba, Wb, bb, g, b, out, zbuf, stats):
    i = pl.program_id(0)

    @pl.when(i == 0)
    def _():
        stats[...] = jnp.zeros_like(stats)

    @pl.when(i < NB)
    def _():
        _layer_compute(h0, h1, a0, a1, Wa, ba, Wb, bb, zbuf, stats, i)

    @pl.when((i >= NB) & (i < 2 * NB))
    def _():
        scale, shift = _bn_coeffs(stats, g, b)
        zn = zbuf[pl.ds((i - NB) * BN_, BN_), :] * scale + shift
        out[...] = zn[:, :HN]

    @pl.when(i >= 2 * NB)
    def _():
        scale, shift = _bn_coeffs(stats, g, b)
        zn = zbuf[pl.ds((i - 2 * NB) * BN_, BN_), :] * scale + shift
        out[...] = zn[:, HN:]


def _mlp_last_body(h0, h1, a0, a1, Wa, ba, Wb, bb, g, b, out, zbuf, stats):
    i = pl.program_id(0)

    @pl.when(i == 0)
    def _():
        stats[...] = jnp.zeros_like(stats)

    @pl.when(i < NB)
    def _():
        _layer_compute(h0, h1, a0, a1, Wa, ba, Wb, bb, zbuf, stats, i)

    @pl.when(i >= NB)
    def _():
        scale, shift = _bn_coeffs(stats, g, b)
        out[...] = zbuf[pl.ds((i - NB) * BN_, BN_), :] * scale + shift


def _clampi(i):
    return jnp.minimum(i, NB - 1)


_COMMON_IN_SPECS = [
    pl.BlockSpec((BN_, HN), lambda i: (_clampi(i), 0)),
    pl.BlockSpec((BN_, HN), lambda i: (NB + _clampi(i), 0)),
    pl.BlockSpec((BN_, HN), lambda i: (_clampi(i), 0)),
    pl.BlockSpec((BN_, HN), lambda i: (NB + _clampi(i), 0)),
    pl.BlockSpec((H, H), lambda i: (0, 0)),
    pl.BlockSpec((1, H), lambda i: (0, 0)),
    pl.BlockSpec((H, H), lambda i: (0, 0)),
    pl.BlockSpec((1, H), lambda i: (0, 0)),
    pl.BlockSpec((1, H), lambda i: (0, 0)),
    pl.BlockSpec((1, H), lambda i: (0, 0)),
]

_mlp_mid = pl.pallas_call(
    _mlp_mid_body,
    grid=(3 * NB,),
    in_specs=_COMMON_IN_SPECS,
    out_specs=pl.BlockSpec((BN_, HN), lambda i: (jnp.maximum(i - NB, 0), 0)),
    out_shape=jax.ShapeDtypeStruct((2 * N, HN), jnp.float32),
    scratch_shapes=[pltpu.VMEM((N, H), jnp.float32),
                    pltpu.VMEM((8, H), jnp.float32)],
    compiler_params=pltpu.CompilerParams(
        dimension_semantics=("arbitrary",)),
)

_mlp_last = pl.pallas_call(
    _mlp_last_body,
    grid=(2 * NB,),
    in_specs=_COMMON_IN_SPECS,
    out_specs=pl.BlockSpec((BN_, H), lambda i: (jnp.maximum(i - NB, 0), 0)),
    out_shape=jax.ShapeDtypeStruct((N, H), jnp.float32),
    scratch_shapes=[pltpu.VMEM((N, H), jnp.float32),
                    pltpu.VMEM((8, H), jnp.float32)],
    compiler_params=pltpu.CompilerParams(
        dimension_semantics=("arbitrary",)),
)


def _pool_head_body(hf, batch, W1, b1, W2, b2, out, pooled):
    i = pl.program_id(0)

    @pl.when(i == 0)
    def _():
        pooled[...] = jnp.zeros_like(pooled)

    oh = (batch[...] == lax.broadcasted_iota(jnp.int32, (BN_, G), 1))
    oh = oh.astype(jnp.float32)
    zaug = jnp.concatenate(
        [hf[...], jnp.ones((BN_, HN), jnp.float32)], axis=1)
    pooled[...] += lax.dot_general(oh, zaug, (((0,), (0,)), ((), ())),
                                   preferred_element_type=jnp.float32)

    @pl.when(i == NB - 1)
    def _():
        P = pooled[...]
        cnt = P[:, H:H + 1]
        pm = P[:, :H] / jnp.maximum(cnt, 1.0)
        o = jnp.maximum(_mm(pm, W1[...]) + b1[...], 0.0)
        o = _mm(o, W2[...]) + b2[...]
        m = jnp.max(o, axis=1, keepdims=True)
        lse = jnp.log(jnp.sum(jnp.exp(o - m), axis=1, keepdims=True))
        out[...] = o - m - lse


_pool_head = pl.pallas_call(
    _pool_head_body,
    grid=(NB,),
    in_specs=[
        pl.BlockSpec((BN_, H), lambda i: (i, 0)),
        pl.BlockSpec((BN_, 1), lambda i: (i, 0)),
        pl.BlockSpec((H, H), lambda i: (0, 0)),
        pl.BlockSpec((1, H), lambda i: (0, 0)),
        pl.BlockSpec((H, C), lambda i: (0, 0)),
        pl.BlockSpec((1, C), lambda i: (0, 0)),
    ],
    out_specs=pl.BlockSpec((G, C), lambda i: (0, 0)),
    out_shape=jax.ShapeDtypeStruct((G, C), jnp.float32),
    scratch_shapes=[pltpu.VMEM((G, H + HN), jnp.float32)],
    compiler_params=pltpu.CompilerParams(
        dimension_semantics=("arbitrary",)),
)


def kernel(x, edge_index, batch, Wa, ba, Wb, bb, gamma, beta, W1, b1, W2, b2):
    src = edge_index[0].astype(jnp.int32)
    dst = edge_index[1].astype(jnp.int32)
    src2 = jnp.concatenate([src, src + N])
    dst2 = dst
    zer = jnp.zeros((RPT_LAST, HN), jnp.float32)
    batch2 = batch.reshape(N, 1).astype(jnp.int32)
    h2 = x.reshape(N, 2, HN).transpose(1, 0, 2).reshape(2 * N, HN)
    seg_sum = _get_seg_sum()
    for l in range(NL):
        agg2 = seg_sum(h2, src2, dst2, zer)
        args = (h2, h2, agg2, agg2, Wa[l], ba[l].reshape(1, H), Wb[l],
                bb[l].reshape(1, H), gamma[l].reshape(1, H),
                beta[l].reshape(1, H))
        if l < NL - 1:
            h2 = _mlp_mid(*args)
        else:
            hf = _mlp_last(*args)
    return _pool_head(hf, batch2, W1, b1.reshape(1, H), W2,
                      b2.reshape(1, C))


# speedup vs baseline: 1.5093x; 1.0258x over previous
import jax
import jax.numpy as jnp
from jax import lax
from jax.experimental import pallas as pl
from jax.experimental.pallas import tpu as pltpu
from jax.experimental.pallas import tpu_sc as plsc

N = 10000
E = 160000
H = 256
NL = 5
G = 64
C = 10

NC = 2
NS = 16
HN = H // 2
EPT = E // NS
K = 80
NCHUNK = EPT // K
RPT = 624
RPT_LAST = N - (NS - 1) * RPT

BN_ = 2000
NB = N // BN_


NSI = 8
NRW = 4


def _seg_sum_body(h2, src2, dst2, zer, out,
                  si0, si1, si2, si3, si4, si5, si6, si7,
                  dd0, dd1, dd2, dd3,
                  rw0, rw1, rw2, rw3, acc,
                  is0, is1, is2, is3, is4, is5, is6, is7,
                  dsm0, dsm1, dsm2, dsm3,
                  gs0, gs1, gs2, gs3, ss0, ss1, ss2, ss3):
    sis = [si0, si1, si2, si3, si4, si5, si6, si7]
    dds = [dd0, dd1, dd2, dd3]
    rws = [rw0, rw1, rw2, rw3]
    isem = [is0, is1, is2, is3, is4, is5, is6, is7]
    dsem = [dsm0, dsm1, dsm2, dsm3]
    gsem = [gs0, gs1, gs2, gs3]
    ssem = [ss0, ss1, ss2, ss3]
    c = lax.axis_index("c")
    s = lax.axis_index("s")
    ebase = c * E + s * EPT
    dbase = s * EPT

    def iload(j, q):
        pltpu.async_copy(src2.at[pl.ds(ebase + j * K, K)], sis[q], isem[q])

    def dload(j, b):
        pltpu.async_copy(dst2.at[pl.ds(dbase + j * K, K)], dds[b], dsem[b])

    def gather(q, b):
        pltpu.async_copy(h2.at[sis[q]], rws[b], gsem[b])

    def wait_iload(q):
        pltpu.make_async_copy(src2.at[pl.ds(dbase, K)], sis[q],
                              isem[q]).wait()

    def wait_dload(b):
        pltpu.make_async_copy(dst2.at[pl.ds(dbase, K)], dds[b],
                              dsem[b]).wait()

    def wait_gather(b):
        pltpu.make_async_copy(h2.at[sis[b]], rws[b], gsem[b]).wait()

    def wait_scatter(b):
        pltpu.make_async_copy(rws[b], acc.at[dds[b]], ssem[b]).wait()

    @pl.when(s < NS - 1)
    def _():
        pltpu.sync_copy(zer.at[pl.ds(0, RPT)], acc.at[pl.ds(s * RPT, RPT)])

    @pl.when(s == NS - 1)
    def _():
        pltpu.sync_copy(zer, acc.at[pl.ds((NS - 1) * RPT, RPT_LAST)])

    plsc.subcore_barrier()

    for q in range(NSI):
        iload(q, q)
    for b in range(NRW):
        dload(b, b)
    for b in range(NRW):
        wait_iload(b)
        gather(b, b)

    GRP = NCHUNK // NSI
    TAIL = GRP * NSI

    def emit(j, u, in_loop):
        b, q = u % NRW, u
        wait_gather(b)
        wait_dload(b)
        pltpu.async_copy(rws[b], acc.at[dds[b]], ssem[b], add=True)
        nxt_load = j + NSI < NCHUNK
        nxt_gather = j + NRW < NCHUNK

        def advance():
            wait_scatter(b)
            dload(j + NRW, b)
            wait_iload((u + NRW) % NSI)
            gather((u + NRW) % NSI, b)

        if in_loop:
            @pl.when(nxt_load)
            def _():
                iload(j + NSI, q)

            @pl.when(nxt_gather)
            def _():
                advance()
        else:
            if nxt_load:
                iload(j + NSI, q)
            if nxt_gather:
                advance()

    def body(g, carry):
        for u in range(NSI):
            emit(NSI * g + u, u, True)
        return carry

    lax.fori_loop(0, GRP, body, 0)
    for j in range(TAIL, NCHUNK):
        emit(j, j % NSI, False)
    for b in range(NRW):
        wait_scatter(b)

    plsc.subcore_barrier()

    @pl.when(s < NS - 1)
    def _():
        pltpu.sync_copy(acc.at[pl.ds(s * RPT, RPT)],
                        out.at[pl.ds(c * N + s * RPT, RPT)])

    @pl.when(s == NS - 1)
    def _():
        pltpu.sync_copy(acc.at[pl.ds((NS - 1) * RPT, RPT_LAST)],
                        out.at[pl.ds(c * N + (NS - 1) * RPT, RPT_LAST)])


_seg_sum_cache = None


def _get_seg_sum():
    global _seg_sum_cache
    if _seg_sum_cache is None:
        _seg_sum_cache = pl.kernel(
            _seg_sum_body,
            out_type=jax.ShapeDtypeStruct((2 * N, HN), jnp.float32),
            mesh=plsc.VectorSubcoreMesh(core_axis_name="c",
                                        subcore_axis_name="s",
                                        num_cores=NC, num_subcores=NS),
            scratch_types=(
                [pltpu.VMEM((K,), jnp.int32) for _ in range(NSI)]
                + [pltpu.VMEM((K,), jnp.int32) for _ in range(NRW)]
                + [pltpu.VMEM((K, HN), jnp.float32) for _ in range(NRW)]
                + [pltpu.VMEM_SHARED((N, HN), jnp.float32)]
                + [pltpu.SemaphoreType.DMA
                   for _ in range(NSI + 3 * NRW)]
            ),
        )
    return _seg_sum_cache


def _mm(a, b):
    return lax.dot_general(a, b, (((1,), (0,)), ((), ())),
                           preferred_element_type=jnp.float32)


def _layer_compute(h0, h1, a0, a1, Wa, ba, Wb, bb, zbuf, stats, i):
    u = jnp.concatenate([h0[...] + a0[...], h1[...] + a1[...]], axis=1)
    z = jnp.maximum(_mm(u, Wa[...]) + ba[...], 0.0)
    z = jnp.maximum(_mm(z, Wb[...]) + bb[...], 0.0)
    zbuf[pl.ds(i * BN_, BN_), :] = z
    stats[0:1, :] += jnp.sum(z, axis=0, keepdims=True)
    stats[1:2, :] += jnp.sum(z * z, axis=0, keepdims=True)


def _bn_coeffs(stats, g, b):
    mu = stats[0:1, :] * (1.0 / N)
    var = stats[1:2, :] * (1.0 / N) - mu * mu
    scale = g[...] * lax.rsqrt(var + 1e-5)
    shift = b[...] - mu * scale
    return scale, shift


def _mlp_mid_body(h0, h1, a0, a1, Wa, ba, Wb, bb, g, b, out, zbuf, stats):
    i = pl.program_id(0)

    @pl.when(i == 0)
    def _():
        stats[...] = jnp.zeros_like(stats)

    @pl.when(i < NB)
    def _():
        _layer_compute(h0, h1, a0, a1, Wa, ba, Wb, bb, zbuf, stats, i)

    @pl.when((i >= NB) & (i < 2 * NB))
    def _():
        scale, shift = _bn_coeffs(stats, g, b)
        zn = zbuf[pl.ds((i - NB) * BN_, BN_), :] * scale + shift
        out[...] = zn[:, :HN]

    @pl.when(i >= 2 * NB)
    def _():
        scale, shift = _bn_coeffs(stats, g, b)
        zn = zbuf[pl.ds((i - 2 * NB) * BN_, BN_), :] * scale + shift
        out[...] = zn[:, HN:]


def _mlp_last_body(h0, h1, a0, a1, Wa, ba, Wb, bb, g, b, out, zbuf, stats):
    i = pl.program_id(0)

    @pl.when(i == 0)
    def _():
        stats[...] = jnp.zeros_like(stats)

    @pl.when(i < NB)
    def _():
        _layer_compute(h0, h1, a0, a1, Wa, ba, Wb, bb, zbuf, stats, i)

    @pl.when(i >= NB)
    def _():
        scale, shift = _bn_coeffs(stats, g, b)
        out[...] = zbuf[pl.ds((i - NB) * BN_, BN_), :] * scale + shift


def _clampi(i):
    return jnp.minimum(i, NB - 1)


_COMMON_IN_SPECS = [
    pl.BlockSpec((BN_, HN), lambda i: (_clampi(i), 0)),
    pl.BlockSpec((BN_, HN), lambda i: (NB + _clampi(i), 0)),
    pl.BlockSpec((BN_, HN), lambda i: (_clampi(i), 0)),
    pl.BlockSpec((BN_, HN), lambda i: (NB + _clampi(i), 0)),
    pl.BlockSpec((H, H), lambda i: (0, 0)),
    pl.BlockSpec((1, H), lambda i: (0, 0)),
    pl.BlockSpec((H, H), lambda i: (0, 0)),
    pl.BlockSpec((1, H), lambda i: (0, 0)),
    pl.BlockSpec((1, H), lambda i: (0, 0)),
    pl.BlockSpec((1, H), lambda i: (0, 0)),
]

_mlp_mid = pl.pallas_call(
    _mlp_mid_body,
    grid=(3 * NB,),
    in_specs=_COMMON_IN_SPECS,
    out_specs=pl.BlockSpec((BN_, HN), lambda i: (jnp.maximum(i - NB, 0), 0)),
    out_shape=jax.ShapeDtypeStruct((2 * N, HN), jnp.float32),
    scratch_shapes=[pltpu.VMEM((N, H), jnp.float32),
                    pltpu.VMEM((8, H), jnp.float32)],
    compiler_params=pltpu.CompilerParams(
        dimension_semantics=("arbitrary",)),
)

_mlp_last = pl.pallas_call(
    _mlp_last_body,
    grid=(2 * NB,),
    in_specs=_COMMON_IN_SPECS,
    out_specs=pl.BlockSpec((BN_, H), lambda i: (jnp.maximum(i - NB, 0), 0)),
    out_shape=jax.ShapeDtypeStruct((N, H), jnp.float32),
    scratch_shapes=[pltpu.VMEM((N, H), jnp.float32),
                    pltpu.VMEM((8, H), jnp.float32)],
    compiler_params=pltpu.CompilerParams(
        dimension_semantics=("arbitrary",)),
)


def _pool_head_body(hf, batch, W1, b1, W2, b2, out, pooled):
    i = pl.program_id(0)

    @pl.when(i == 0)
    def _():
        pooled[...] = jnp.zeros_like(pooled)

    oh = (batch[...] == lax.broadcasted_iota(jnp.int32, (BN_, G), 1))
    oh = oh.astype(jnp.float32)
    zaug = jnp.concatenate(
        [hf[...], jnp.ones((BN_, HN), jnp.float32)], axis=1)
    pooled[...] += lax.dot_general(oh, zaug, (((0,), (0,)), ((), ())),
                                   preferred_element_type=jnp.float32)

    @pl.when(i == NB - 1)
    def _():
        P = pooled[...]
        cnt = P[:, H:H + 1]
        pm = P[:, :H] / jnp.maximum(cnt, 1.0)
        o = jnp.maximum(_mm(pm, W1[...]) + b1[...], 0.0)
        o = _mm(o, W2[...]) + b2[...]
        m = jnp.max(o, axis=1, keepdims=True)
        lse = jnp.log(jnp.sum(jnp.exp(o - m), axis=1, keepdims=True))
        out[...] = o - m - lse


_pool_head = pl.pallas_call(
    _pool_head_body,
    grid=(NB,),
    in_specs=[
        pl.BlockSpec((BN_, H), lambda i: (i, 0)),
        pl.BlockSpec((BN_, 1), lambda i: (i, 0)),
        pl.BlockSpec((H, H), lambda i: (0, 0)),
        pl.BlockSpec((1, H), lambda i: (0, 0)),
        pl.BlockSpec((H, C), lambda i: (0, 0)),
        pl.BlockSpec((1, C), lambda i: (0, 0)),
    ],
    out_specs=pl.BlockSpec((G, C), lambda i: (0, 0)),
    out_shape=jax.ShapeDtypeStruct((G, C), jnp.float32),
    scratch_shapes=[pltpu.VMEM((G, H + HN), jnp.float32)],
    compiler_params=pltpu.CompilerParams(
        dimension_semantics=("arbitrary",)),
)


def kernel(x, edge_index, batch, Wa, ba, Wb, bb, gamma, beta, W1, b1, W2, b2):
    src = edge_index[0].astype(jnp.int32)
    dst = edge_index[1].astype(jnp.int32)
    src2 = jnp.concatenate([src, src + N])
    dst2 = dst
    zer = jnp.zeros((RPT_LAST, HN), jnp.float32)
    batch2 = batch.reshape(N, 1).astype(jnp.int32)
    h2 = x.reshape(N, 2, HN).transpose(1, 0, 2).reshape(2 * N, HN)
    seg_sum = _get_seg_sum()
    for l in range(NL):
        agg2 = seg_sum(h2, src2, dst2, zer)
        args = (h2, h2, agg2, agg2, Wa[l], ba[l].reshape(1, H), Wb[l],
                bb[l].reshape(1, H), gamma[l].reshape(1, H),
                beta[l].reshape(1, H))
        if l < NL - 1:
            h2 = _mlp_mid(*args)
        else:
            hf = _mlp_last(*args)
    return _pool_head(hf, batch2, W1, b1.reshape(1, H), W2,
                      b2.reshape(1, C))


# speedup vs baseline: 1.5756x; 1.0440x over previous
import jax
import jax.numpy as jnp
from jax import lax
from jax.experimental import pallas as pl
from jax.experimental.pallas import tpu as pltpu
from jax.experimental.pallas import tpu_sc as plsc

N = 10000
E = 160000
H = 256
NL = 5
G = 64
C = 10

NC = 2
NS = 16
HN = H // 2
EPT = E // NS
K = 80
NCHUNK = EPT // K
RPT = 624
RPT_LAST = N - (NS - 1) * RPT

BN_ = 2000
NB = N // BN_


NSI = 8
NRW = 4


def _seg_sum_body(h2, src2, dst2, out,
                  si0, si1, si2, si3, si4, si5, si6, si7,
                  dd0, dd1, dd2, dd3,
                  rw0, rw1, rw2, rw3, acc,
                  is0, is1, is2, is3, is4, is5, is6, is7,
                  dsm0, dsm1, dsm2, dsm3,
                  gs0, gs1, gs2, gs3, ss0, ss1, ss2, ss3):
    sis = [si0, si1, si2, si3, si4, si5, si6, si7]
    dds = [dd0, dd1, dd2, dd3]
    rws = [rw0, rw1, rw2, rw3]
    isem = [is0, is1, is2, is3, is4, is5, is6, is7]
    dsem = [dsm0, dsm1, dsm2, dsm3]
    gsem = [gs0, gs1, gs2, gs3]
    ssem = [ss0, ss1, ss2, ss3]
    c = lax.axis_index("c")
    s = lax.axis_index("s")
    ebase = c * E + s * EPT
    dbase = s * EPT

    def iload(j, q):
        pltpu.async_copy(src2.at[pl.ds(ebase + j * K, K)], sis[q], isem[q])

    def dload(j, b):
        pltpu.async_copy(dst2.at[pl.ds(dbase + j * K, K)], dds[b], dsem[b])

    def gather(q, b):
        pltpu.async_copy(h2.at[sis[q]], rws[b], gsem[b])

    def wait_iload(q):
        pltpu.make_async_copy(src2.at[pl.ds(dbase, K)], sis[q],
                              isem[q]).wait()

    def wait_dload(b):
        pltpu.make_async_copy(dst2.at[pl.ds(dbase, K)], dds[b],
                              dsem[b]).wait()

    def wait_gather(b):
        pltpu.make_async_copy(h2.at[sis[b]], rws[b], gsem[b]).wait()

    def wait_scatter(b):
        pltpu.make_async_copy(rws[b], acc.at[dds[b]], ssem[b]).wait()

    @pl.when(s < NS - 1)
    def _():
        pltpu.sync_copy(h2.at[pl.ds(c * N + s * RPT, RPT)],
                        acc.at[pl.ds(s * RPT, RPT)])

    @pl.when(s == NS - 1)
    def _():
        pltpu.sync_copy(h2.at[pl.ds(c * N + (NS - 1) * RPT, RPT_LAST)],
                        acc.at[pl.ds((NS - 1) * RPT, RPT_LAST)])

    plsc.subcore_barrier()

    for q in range(NSI):
        iload(q, q)
    for b in range(NRW):
        dload(b, b)
    for b in range(NRW):
        wait_iload(b)
        gather(b, b)

    GRP = NCHUNK // NSI
    TAIL = GRP * NSI

    def emit(j, u, in_loop):
        b, q = u % NRW, u
        wait_gather(b)
        wait_dload(b)
        pltpu.async_copy(rws[b], acc.at[dds[b]], ssem[b], add=True)
        nxt_load = j + NSI < NCHUNK
        nxt_gather = j + NRW < NCHUNK

        def advance():
            wait_scatter(b)
            dload(j + NRW, b)
            wait_iload((u + NRW) % NSI)
            gather((u + NRW) % NSI, b)

        if in_loop:
            @pl.when(nxt_load)
            def _():
                iload(j + NSI, q)

            @pl.when(nxt_gather)
            def _():
                advance()
        else:
            if nxt_load:
                iload(j + NSI, q)
            if nxt_gather:
                advance()

    def body(g, carry):
        for u in range(NSI):
            emit(NSI * g + u, u, True)
        return carry

    lax.fori_loop(0, GRP, body, 0)
    for j in range(TAIL, NCHUNK):
        emit(j, j % NSI, False)
    for b in range(NRW):
        wait_scatter(b)

    plsc.subcore_barrier()

    @pl.when(s < NS - 1)
    def _():
        pltpu.sync_copy(acc.at[pl.ds(s * RPT, RPT)],
                        out.at[pl.ds(c * N + s * RPT, RPT)])

    @pl.when(s == NS - 1)
    def _():
        pltpu.sync_copy(acc.at[pl.ds((NS - 1) * RPT, RPT_LAST)],
                        out.at[pl.ds(c * N + (NS - 1) * RPT, RPT_LAST)])


_seg_sum_cache = None


def _get_seg_sum():
    global _seg_sum_cache
    if _seg_sum_cache is None:
        _seg_sum_cache = pl.kernel(
            _seg_sum_body,
            out_type=jax.ShapeDtypeStruct((2 * N, HN), jnp.float32),
            mesh=plsc.VectorSubcoreMesh(core_axis_name="c",
                                        subcore_axis_name="s",
                                        num_cores=NC, num_subcores=NS),
            scratch_types=(
                [pltpu.VMEM((K,), jnp.int32) for _ in range(NSI)]
                + [pltpu.VMEM((K,), jnp.int32) for _ in range(NRW)]
                + [pltpu.VMEM((K, HN), jnp.float32) for _ in range(NRW)]
                + [pltpu.VMEM_SHARED((N, HN), jnp.float32)]
                + [pltpu.SemaphoreType.DMA
                   for _ in range(NSI + 3 * NRW)]
            ),
        )
    return _seg_sum_cache


def _mm(a, b):
    return lax.dot_general(a, b, (((1,), (0,)), ((), ())),
                           preferred_element_type=jnp.float32)


def _layer_compute(a0, a1, Wa, ba, Wb, bb, zbuf, stats, i):
    u = jnp.concatenate([a0[...], a1[...]], axis=1)
    z = jnp.maximum(_mm(u, Wa[...]) + ba[...], 0.0)
    z = jnp.maximum(_mm(z, Wb[...]) + bb[...], 0.0)
    zbuf[pl.ds(i * BN_, BN_), :] = z
    stats[0:1, :] += jnp.sum(z, axis=0, keepdims=True)
    stats[1:2, :] += jnp.sum(z * z, axis=0, keepdims=True)


def _bn_coeffs(stats, g, b):
    mu = stats[0:1, :] * (1.0 / N)
    var = stats[1:2, :] * (1.0 / N) - mu * mu
    scale = g[...] * lax.rsqrt(var + 1e-5)
    shift = b[...] - mu * scale
    return scale, shift


def _mlp_mid_body(a0, a1, Wa, ba, Wb, bb, g, b, out, zbuf, stats):
    i = pl.program_id(0)

    @pl.when(i == 0)
    def _():
        stats[...] = jnp.zeros_like(stats)

    @pl.when(i < NB)
    def _():
        _layer_compute(a0, a1, Wa, ba, Wb, bb, zbuf, stats, i)

    @pl.when((i >= NB) & (i < 2 * NB))
    def _():
        scale, shift = _bn_coeffs(stats, g, b)
        zn = zbuf[pl.ds((i - NB) * BN_, BN_), :] * scale + shift
        out[...] = zn[:, :HN]

    @pl.when(i >= 2 * NB)
    def _():
        scale, shift = _bn_coeffs(stats, g, b)
        zn = zbuf[pl.ds((i - 2 * NB) * BN_, BN_), :] * scale + shift
        out[...] = zn[:, HN:]


def _mlp_last_body(a0, a1, Wa, ba, Wb, bb, g, b, out, zbuf, stats):
    i = pl.program_id(0)

    @pl.when(i == 0)
    def _():
        stats[...] = jnp.zeros_like(stats)

    @pl.when(i < NB)
    def _():
        _layer_compute(a0, a1, Wa, ba, Wb, bb, zbuf, stats, i)

    @pl.when(i >= NB)
    def _():
        scale, shift = _bn_coeffs(stats, g, b)
        out[...] = zbuf[pl.ds((i - NB) * BN_, BN_), :] * scale + shift


def _clampi(i):
    return jnp.minimum(i, NB - 1)


_COMMON_IN_SPECS = [
    pl.BlockSpec((BN_, HN), lambda i: (_clampi(i), 0)),
    pl.BlockSpec((BN_, HN), lambda i: (NB + _clampi(i), 0)),
    pl.BlockSpec((H, H), lambda i: (0, 0)),
    pl.BlockSpec((1, H), lambda i: (0, 0)),
    pl.BlockSpec((H, H), lambda i: (0, 0)),
    pl.BlockSpec((1, H), lambda i: (0, 0)),
    pl.BlockSpec((1, H), lambda i: (0, 0)),
    pl.BlockSpec((1, H), lambda i: (0, 0)),
]

_mlp_mid = pl.pallas_call(
    _mlp_mid_body,
    grid=(3 * NB,),
    in_specs=_COMMON_IN_SPECS,
    out_specs=pl.BlockSpec((BN_, HN), lambda i: (jnp.maximum(i - NB, 0), 0)),
    out_shape=jax.ShapeDtypeStruct((2 * N, HN), jnp.float32),
    scratch_shapes=[pltpu.VMEM((N, H), jnp.float32),
                    pltpu.VMEM((8, H), jnp.float32)],
    compiler_params=pltpu.CompilerParams(
        dimension_semantics=("arbitrary",)),
)

_mlp_last = pl.pallas_call(
    _mlp_last_body,
    grid=(2 * NB,),
    in_specs=_COMMON_IN_SPECS,
    out_specs=pl.BlockSpec((BN_, H), lambda i: (jnp.maximum(i - NB, 0), 0)),
    out_shape=jax.ShapeDtypeStruct((N, H), jnp.float32),
    scratch_shapes=[pltpu.VMEM((N, H), jnp.float32),
                    pltpu.VMEM((8, H), jnp.float32)],
    compiler_params=pltpu.CompilerParams(
        dimension_semantics=("arbitrary",)),
)


def _pool_head_body(hf, batch, W1, b1, W2, b2, out, pooled):
    i = pl.program_id(0)

    @pl.when(i == 0)
    def _():
        pooled[...] = jnp.zeros_like(pooled)

    oh = (batch[...] == lax.broadcasted_iota(jnp.int32, (BN_, G), 1))
    oh = oh.astype(jnp.float32)
    zaug = jnp.concatenate(
        [hf[...], jnp.ones((BN_, HN), jnp.float32)], axis=1)
    pooled[...] += lax.dot_general(oh, zaug, (((0,), (0,)), ((), ())),
                                   preferred_element_type=jnp.float32)

    @pl.when(i == NB - 1)
    def _():
        P = pooled[...]
        cnt = P[:, H:H + 1]
        pm = P[:, :H] / jnp.maximum(cnt, 1.0)
        o = jnp.maximum(_mm(pm, W1[...]) + b1[...], 0.0)
        o = _mm(o, W2[...]) + b2[...]
        m = jnp.max(o, axis=1, keepdims=True)
        lse = jnp.log(jnp.sum(jnp.exp(o - m), axis=1, keepdims=True))
        out[...] = o - m - lse


_pool_head = pl.pallas_call(
    _pool_head_body,
    grid=(NB,),
    in_specs=[
        pl.BlockSpec((BN_, H), lambda i: (i, 0)),
        pl.BlockSpec((BN_, 1), lambda i: (i, 0)),
        pl.BlockSpec((H, H), lambda i: (0, 0)),
        pl.BlockSpec((1, H), lambda i: (0, 0)),
        pl.BlockSpec((H, C), lambda i: (0, 0)),
        pl.BlockSpec((1, C), lambda i: (0, 0)),
    ],
    out_specs=pl.BlockSpec((G, C), lambda i: (0, 0)),
    out_shape=jax.ShapeDtypeStruct((G, C), jnp.float32),
    scratch_shapes=[pltpu.VMEM((G, H + HN), jnp.float32)],
    compiler_params=pltpu.CompilerParams(
        dimension_semantics=("arbitrary",)),
)


def kernel(x, edge_index, batch, Wa, ba, Wb, bb, gamma, beta, W1, b1, W2, b2):
    src = edge_index[0].astype(jnp.int32)
    dst = edge_index[1].astype(jnp.int32)
    src2 = jnp.concatenate([src, src + N])
    dst2 = dst
    batch2 = batch.reshape(N, 1).astype(jnp.int32)
    h2 = x.reshape(N, 2, HN).transpose(1, 0, 2).reshape(2 * N, HN)
    seg_sum = _get_seg_sum()
    for l in range(NL):
        agg2 = seg_sum(h2, src2, dst2)
        args = (agg2, agg2, Wa[l], ba[l].reshape(1, H), Wb[l],
                bb[l].reshape(1, H), gamma[l].reshape(1, H),
                beta[l].reshape(1, H))
        if l < NL - 1:
            h2 = _mlp_mid(*args)
        else:
            hf = _mlp_last(*args)
    return _pool_head(hf, batch2, W1, b1.reshape(1, H), W2,
                      b2.reshape(1, C))


# speedup vs baseline: 1.6261x; 1.0320x over previous
import jax
import jax.numpy as jnp
from jax import lax
from jax.experimental import pallas as pl
from jax.experimental.pallas import tpu as pltpu
from jax.experimental.pallas import tpu_sc as plsc

N = 10000
E = 160000
H = 256
NL = 5
G = 64
C = 10

NC = 2
NS = 16
HN = H // 2
EPT = E // NS
K = 80
NCHUNK = EPT // K
RPT = 624
RPT_LAST = N - (NS - 1) * RPT

BN_ = 2000
NB = N // BN_


NSI = 8
NRW = 4


def _seg_sum_body(h2, src2, dst2, out,
                  si0, si1, si2, si3, si4, si5, si6, si7,
                  dd0, dd1, dd2, dd3,
                  rw0, rw1, rw2, rw3, acc,
                  is0, is1, is2, is3, is4, is5, is6, is7,
                  dsm0, dsm1, dsm2, dsm3,
                  gs0, gs1, gs2, gs3, ss0, ss1, ss2, ss3):
    sis = [si0, si1, si2, si3, si4, si5, si6, si7]
    dds = [dd0, dd1, dd2, dd3]
    rws = [rw0, rw1, rw2, rw3]
    isem = [is0, is1, is2, is3, is4, is5, is6, is7]
    dsem = [dsm0, dsm1, dsm2, dsm3]
    gsem = [gs0, gs1, gs2, gs3]
    ssem = [ss0, ss1, ss2, ss3]
    c = lax.axis_index("c")
    s = lax.axis_index("s")
    ebase = c * E + s * EPT
    dbase = s * EPT

    def iload(j, q):
        pltpu.async_copy(src2.at[pl.ds(ebase + j * K, K)], sis[q], isem[q])

    def dload(j, b):
        pltpu.async_copy(dst2.at[pl.ds(dbase + j * K, K)], dds[b], dsem[b])

    def gather(q, b):
        pltpu.async_copy(h2.at[sis[q]], rws[b], gsem[b])

    def wait_iload(q):
        pltpu.make_async_copy(src2.at[pl.ds(dbase, K)], sis[q],
                              isem[q]).wait()

    def wait_dload(b):
        pltpu.make_async_copy(dst2.at[pl.ds(dbase, K)], dds[b],
                              dsem[b]).wait()

    def wait_gather(b):
        pltpu.make_async_copy(h2.at[sis[b]], rws[b], gsem[b]).wait()

    def wait_scatter(b):
        pltpu.make_async_copy(rws[b], acc.at[dds[b]], ssem[b]).wait()

    for q in range(NSI):
        iload(q, q)
    for b in range(NRW):
        dload(b, b)

    @pl.when(s < NS - 1)
    def _():
        pltpu.sync_copy(h2.at[pl.ds(c * N + s * RPT, RPT)],
                        acc.at[pl.ds(s * RPT, RPT)])

    @pl.when(s == NS - 1)
    def _():
        pltpu.sync_copy(h2.at[pl.ds(c * N + (NS - 1) * RPT, RPT_LAST)],
                        acc.at[pl.ds((NS - 1) * RPT, RPT_LAST)])

    for b in range(NRW):
        wait_iload(b)
        gather(b, b)

    plsc.subcore_barrier()

    GRP = NCHUNK // NSI
    TAIL = GRP * NSI

    def emit(j, u, in_loop):
        b, q = u % NRW, u
        wait_gather(b)
        wait_dload(b)
        pltpu.async_copy(rws[b], acc.at[dds[b]], ssem[b], add=True)
        nxt_load = j + NSI < NCHUNK
        nxt_gather = j + NRW < NCHUNK

        def advance():
            wait_scatter(b)
            dload(j + NRW, b)
            wait_iload((u + NRW) % NSI)
            gather((u + NRW) % NSI, b)

        if in_loop:
            @pl.when(nxt_load)
            def _():
                iload(j + NSI, q)

            @pl.when(nxt_gather)
            def _():
                advance()
        else:
            if nxt_load:
                iload(j + NSI, q)
            if nxt_gather:
                advance()

    def body(g, carry):
        for u in range(NSI):
            emit(NSI * g + u, u, True)
        return carry

    lax.fori_loop(0, GRP, body, 0)
    for j in range(TAIL, NCHUNK):
        emit(j, j % NSI, False)
    for b in range(NRW):
        wait_scatter(b)

    plsc.subcore_barrier()

    @pl.when(s < NS - 1)
    def _():
        pltpu.sync_copy(acc.at[pl.ds(s * RPT, RPT)],
                        out.at[pl.ds(c * N + s * RPT, RPT)])

    @pl.when(s == NS - 1)
    def _():
        pltpu.sync_copy(acc.at[pl.ds((NS - 1) * RPT, RPT_LAST)],
                        out.at[pl.ds(c * N + (NS - 1) * RPT, RPT_LAST)])


_seg_sum_cache = None


def _get_seg_sum():
    global _seg_sum_cache
    if _seg_sum_cache is None:
        _seg_sum_cache = pl.kernel(
            _seg_sum_body,
            out_type=jax.ShapeDtypeStruct((2 * N, HN), jnp.float32),
            mesh=plsc.VectorSubcoreMesh(core_axis_name="c",
                                        subcore_axis_name="s",
                                        num_cores=NC, num_subcores=NS),
            scratch_types=(
                [pltpu.VMEM((K,), jnp.int32) for _ in range(NSI)]
                + [pltpu.VMEM((K,), jnp.int32) for _ in range(NRW)]
                + [pltpu.VMEM((K, HN), jnp.float32) for _ in range(NRW)]
                + [pltpu.VMEM_SHARED((N, HN), jnp.float32)]
                + [pltpu.SemaphoreType.DMA
                   for _ in range(NSI + 3 * NRW)]
            ),
        )
    return _seg_sum_cache


def _mm(a, b):
    return lax.dot_general(a, b, (((1,), (0,)), ((), ())),
                           preferred_element_type=jnp.float32)


def _layer_compute(a0, a1, Wa, ba, Wb, bb, zbuf, stats, i):
    u = jnp.concatenate([a0[...], a1[...]], axis=1)
    z = jnp.maximum(_mm(u, Wa[...]) + ba[...], 0.0)
    z = jnp.maximum(_mm(z, Wb[...]) + bb[...], 0.0)
    zbuf[pl.ds(i * BN_, BN_), :] = z
    stats[0:1, :] += jnp.sum(z, axis=0, keepdims=True)
    stats[1:2, :] += jnp.sum(z * z, axis=0, keepdims=True)


def _bn_coeffs(stats, g, b):
    mu = stats[0:1, :] * (1.0 / N)
    var = stats[1:2, :] * (1.0 / N) - mu * mu
    scale = g[...] * lax.rsqrt(var + 1e-5)
    shift = b[...] - mu * scale
    return scale, shift


def _mlp_mid_body(a0, a1, Wa, ba, Wb, bb, g, b, out, zbuf, stats):
    i = pl.program_id(0)

    @pl.when(i == 0)
    def _():
        stats[...] = jnp.zeros_like(stats)

    @pl.when(i < NB)
    def _():
        _layer_compute(a0, a1, Wa, ba, Wb, bb, zbuf, stats, i)

    @pl.when((i >= NB) & (i < 2 * NB))
    def _():
        scale, shift = _bn_coeffs(stats, g, b)
        zn = zbuf[pl.ds((i - NB) * BN_, BN_), :] * scale + shift
        out[...] = zn[:, :HN]

    @pl.when(i >= 2 * NB)
    def _():
        scale, shift = _bn_coeffs(stats, g, b)
        zn = zbuf[pl.ds((i - 2 * NB) * BN_, BN_), :] * scale + shift
        out[...] = zn[:, HN:]


def _mlp_pool_head_body(a0, a1, batch, Wa, ba, Wb, bb, g, b,
                        W1, b1, W2, b2, out, stats, pooled):
    i = pl.program_id(0)

    @pl.when(i == 0)
    def _():
        stats[...] = jnp.zeros_like(stats)
        pooled[...] = jnp.zeros_like(pooled)

    u = jnp.concatenate([a0[...], a1[...]], axis=1)
    z = jnp.maximum(_mm(u, Wa[...]) + ba[...], 0.0)
    z = jnp.maximum(_mm(z, Wb[...]) + bb[...], 0.0)
    stats[0:1, :] += jnp.sum(z, axis=0, keepdims=True)
    stats[1:2, :] += jnp.sum(z * z, axis=0, keepdims=True)
    oh = (batch[...] == lax.broadcasted_iota(jnp.int32, (BN_, G), 1))
    oh = oh.astype(jnp.float32)
    zaug = jnp.concatenate(
        [z, jnp.ones((BN_, HN), jnp.float32)], axis=1)
    pooled[...] += lax.dot_general(oh, zaug, (((0,), (0,)), ((), ())),
                                   preferred_element_type=jnp.float32)

    @pl.when(i == NB - 1)
    def _():
        scale, shift = _bn_coeffs(stats, g, b)
        P = pooled[...]
        cnt = P[:, H:H + 1]
        pm = (P[:, :H] / jnp.maximum(cnt, 1.0)) * scale + shift
        o = jnp.maximum(_mm(pm, W1[...]) + b1[...], 0.0)
        o = _mm(o, W2[...]) + b2[...]
        m = jnp.max(o, axis=1, keepdims=True)
        lse = jnp.log(jnp.sum(jnp.exp(o - m), axis=1, keepdims=True))
        out[...] = o - m - lse


def _clampi(i):
    return jnp.minimum(i, NB - 1)


_COMMON_IN_SPECS = [
    pl.BlockSpec((BN_, HN), lambda i: (_clampi(i), 0)),
    pl.BlockSpec((BN_, HN), lambda i: (NB + _clampi(i), 0)),
    pl.BlockSpec((H, H), lambda i: (0, 0)),
    pl.BlockSpec((1, H), lambda i: (0, 0)),
    pl.BlockSpec((H, H), lambda i: (0, 0)),
    pl.BlockSpec((1, H), lambda i: (0, 0)),
    pl.BlockSpec((1, H), lambda i: (0, 0)),
    pl.BlockSpec((1, H), lambda i: (0, 0)),
]

_mlp_mid = pl.pallas_call(
    _mlp_mid_body,
    grid=(3 * NB,),
    in_specs=_COMMON_IN_SPECS,
    out_specs=pl.BlockSpec((BN_, HN), lambda i: (jnp.maximum(i - NB, 0), 0)),
    out_shape=jax.ShapeDtypeStruct((2 * N, HN), jnp.float32),
    scratch_shapes=[pltpu.VMEM((N, H), jnp.float32),
                    pltpu.VMEM((8, H), jnp.float32)],
    compiler_params=pltpu.CompilerParams(
        dimension_semantics=("arbitrary",)),
)

_mlp_pool_head = pl.pallas_call(
    _mlp_pool_head_body,
    grid=(NB,),
    in_specs=[
        pl.BlockSpec((BN_, HN), lambda i: (i, 0)),
        pl.BlockSpec((BN_, HN), lambda i: (NB + i, 0)),
        pl.BlockSpec((BN_, 1), lambda i: (i, 0)),
        pl.BlockSpec((H, H), lambda i: (0, 0)),
        pl.BlockSpec((1, H), lambda i: (0, 0)),
        pl.BlockSpec((H, H), lambda i: (0, 0)),
        pl.BlockSpec((1, H), lambda i: (0, 0)),
        pl.BlockSpec((1, H), lambda i: (0, 0)),
        pl.BlockSpec((1, H), lambda i: (0, 0)),
        pl.BlockSpec((H, H), lambda i: (0, 0)),
        pl.BlockSpec((1, H), lambda i: (0, 0)),
        pl.BlockSpec((H, C), lambda i: (0, 0)),
        pl.BlockSpec((1, C), lambda i: (0, 0)),
    ],
    out_specs=pl.BlockSpec((G, C), lambda i: (0, 0)),
    out_shape=jax.ShapeDtypeStruct((G, C), jnp.float32),
    scratch_shapes=[pltpu.VMEM((8, H), jnp.float32),
                    pltpu.VMEM((G, H + HN), jnp.float32)],
    compiler_params=pltpu.CompilerParams(
        dimension_semantics=("arbitrary",)),
)


def kernel(x, edge_index, batch, Wa, ba, Wb, bb, gamma, beta, W1, b1, W2, b2):
    src = edge_index[0].astype(jnp.int32)
    dst = edge_index[1].astype(jnp.int32)
    src2 = jnp.concatenate([src, src + N])
    dst2 = dst
    batch2 = batch.reshape(N, 1).astype(jnp.int32)
    h2 = x.reshape(N, 2, HN).transpose(1, 0, 2).reshape(2 * N, HN)
    seg_sum = _get_seg_sum()
    for l in range(NL - 1):
        agg2 = seg_sum(h2, src2, dst2)
        h2 = _mlp_mid(agg2, agg2, Wa[l], ba[l].reshape(1, H), Wb[l],
                      bb[l].reshape(1, H), gamma[l].reshape(1, H),
                      beta[l].reshape(1, H))
    agg2 = seg_sum(h2, src2, dst2)
    l = NL - 1
    return _mlp_pool_head(agg2, agg2, batch2, Wa[l], ba[l].reshape(1, H),
                          Wb[l], bb[l].reshape(1, H),
                          gamma[l].reshape(1, H), beta[l].reshape(1, H),
                          W1, b1.reshape(1, H), W2, b2.reshape(1, C))


# speedup vs baseline: 1.6407x; 1.0090x over previous
import jax
import jax.numpy as jnp
from jax import lax
from jax.experimental import pallas as pl
from jax.experimental.pallas import tpu as pltpu
from jax.experimental.pallas import tpu_sc as plsc

N = 10000
E = 160000
H = 256
NL = 5
G = 64
C = 10

NC = 2
NS = 16
HN = H // 2
EPT = E // NS
K = 80
NCHUNK = EPT // K
RPT = 624
RPT_LAST = N - (NS - 1) * RPT

BN_ = 2000
NB = N // BN_


NSI = 8
NRW = 4


def _seg_sum_body(h2a, h2b, src2, dst2, out,
                  si0, si1, si2, si3, si4, si5, si6, si7,
                  dd0, dd1, dd2, dd3,
                  rw0, rw1, rw2, rw3, acc,
                  is0, is1, is2, is3, is4, is5, is6, is7,
                  dsm0, dsm1, dsm2, dsm3,
                  gs0, gs1, gs2, gs3, ss0, ss1, ss2, ss3):
    sis = [si0, si1, si2, si3, si4, si5, si6, si7]
    dds = [dd0, dd1, dd2, dd3]
    rws = [rw0, rw1, rw2, rw3]
    isem = [is0, is1, is2, is3, is4, is5, is6, is7]
    dsem = [dsm0, dsm1, dsm2, dsm3]
    gsem = [gs0, gs1, gs2, gs3]
    ssem = [ss0, ss1, ss2, ss3]
    c = lax.axis_index("c")
    s = lax.axis_index("s")
    ebase = s * EPT
    dbase = s * EPT

    def iload(j, q):
        pltpu.async_copy(src2.at[pl.ds(ebase + j * K, K)], sis[q], isem[q])

    def dload(j, b):
        pltpu.async_copy(dst2.at[pl.ds(dbase + j * K, K)], dds[b], dsem[b])

    def gather(q, b):
        @pl.when(c == 0)
        def _():
            pltpu.async_copy(h2a.at[sis[q]], rws[b], gsem[b])

        @pl.when(c == 1)
        def _():
            pltpu.async_copy(h2b.at[sis[q]], rws[b], gsem[b])

    def wait_iload(q):
        pltpu.make_async_copy(src2.at[pl.ds(dbase, K)], sis[q],
                              isem[q]).wait()

    def wait_dload(b):
        pltpu.make_async_copy(dst2.at[pl.ds(dbase, K)], dds[b],
                              dsem[b]).wait()

    def wait_gather(b):
        pltpu.make_async_copy(h2a.at[sis[b]], rws[b], gsem[b]).wait()

    def wait_scatter(b):
        pltpu.make_async_copy(rws[b], acc.at[dds[b]], ssem[b]).wait()

    for q in range(NSI):
        iload(q, q)
    for b in range(NRW):
        dload(b, b)

    @pl.when((c == 0) & (s < NS - 1))
    def _():
        pltpu.sync_copy(h2a.at[pl.ds(s * RPT, RPT)],
                        acc.at[pl.ds(s * RPT, RPT)])

    @pl.when((c == 0) & (s == NS - 1))
    def _():
        pltpu.sync_copy(h2a.at[pl.ds((NS - 1) * RPT, RPT_LAST)],
                        acc.at[pl.ds((NS - 1) * RPT, RPT_LAST)])

    @pl.when((c == 1) & (s < NS - 1))
    def _():
        pltpu.sync_copy(h2b.at[pl.ds(s * RPT, RPT)],
                        acc.at[pl.ds(s * RPT, RPT)])

    @pl.when((c == 1) & (s == NS - 1))
    def _():
        pltpu.sync_copy(h2b.at[pl.ds((NS - 1) * RPT, RPT_LAST)],
                        acc.at[pl.ds((NS - 1) * RPT, RPT_LAST)])

    for b in range(NRW):
        wait_iload(b)
        gather(b, b)

    plsc.subcore_barrier()

    GRP = NCHUNK // NSI
    TAIL = GRP * NSI

    def emit(j, u, in_loop):
        b, q = u % NRW, u
        wait_gather(b)
        wait_dload(b)
        pltpu.async_copy(rws[b], acc.at[dds[b]], ssem[b], add=True)
        nxt_load = j + NSI < NCHUNK
        nxt_gather = j + NRW < NCHUNK

        def advance():
            wait_scatter(b)
            dload(j + NRW, b)
            wait_iload((u + NRW) % NSI)
            gather((u + NRW) % NSI, b)

        if in_loop:
            @pl.when(nxt_load)
            def _():
                iload(j + NSI, q)

            @pl.when(nxt_gather)
            def _():
                advance()
        else:
            if nxt_load:
                iload(j + NSI, q)
            if nxt_gather:
                advance()

    def body(g, carry):
        for u in range(NSI):
            emit(NSI * g + u, u, True)
        return carry

    lax.fori_loop(0, GRP, body, 0)
    for j in range(TAIL, NCHUNK):
        emit(j, j % NSI, False)
    for b in range(NRW):
        wait_scatter(b)

    plsc.subcore_barrier()

    @pl.when(s < NS - 1)
    def _():
        pltpu.sync_copy(acc.at[pl.ds(s * RPT, RPT)],
                        out.at[pl.ds(c * N + s * RPT, RPT)])

    @pl.when(s == NS - 1)
    def _():
        pltpu.sync_copy(acc.at[pl.ds((NS - 1) * RPT, RPT_LAST)],
                        out.at[pl.ds(c * N + (NS - 1) * RPT, RPT_LAST)])


_seg_sum_cache = None


def _get_seg_sum():
    global _seg_sum_cache
    if _seg_sum_cache is None:
        _seg_sum_cache = pl.kernel(
            _seg_sum_body,
            out_type=jax.ShapeDtypeStruct((2 * N, HN), jnp.float32),
            mesh=plsc.VectorSubcoreMesh(core_axis_name="c",
                                        subcore_axis_name="s",
                                        num_cores=NC, num_subcores=NS),
            scratch_types=(
                [pltpu.VMEM((K,), jnp.int32) for _ in range(NSI)]
                + [pltpu.VMEM((K,), jnp.int32) for _ in range(NRW)]
                + [pltpu.VMEM((K, HN), jnp.float32) for _ in range(NRW)]
                + [pltpu.VMEM_SHARED((N, HN), jnp.float32)]
                + [pltpu.SemaphoreType.DMA
                   for _ in range(NSI + 3 * NRW)]
            ),
        )
    return _seg_sum_cache


def _mm(a, b):
    return lax.dot_general(a, b, (((1,), (0,)), ((), ())),
                           preferred_element_type=jnp.float32)


def _layer_compute(a0, a1, Wa, ba, Wb, bb, zbuf, stats, i):
    u = jnp.concatenate([a0[...], a1[...]], axis=1)
    z = jnp.maximum(_mm(u, Wa[...]) + ba[...], 0.0)
    z = jnp.maximum(_mm(z, Wb[...]) + bb[...], 0.0)
    zbuf[pl.ds(i * BN_, BN_), :] = z
    stats[0:1, :] += jnp.sum(z, axis=0, keepdims=True)
    stats[1:2, :] += jnp.sum(z * z, axis=0, keepdims=True)


def _bn_coeffs(stats, g, b):
    mu = stats[0:1, :] * (1.0 / N)
    var = stats[1:2, :] * (1.0 / N) - mu * mu
    scale = g[...] * lax.rsqrt(var + 1e-5)
    shift = b[...] - mu * scale
    return scale, shift


def _mlp_mid_body(a0, a1, Wa, ba, Wb, bb, g, b, out0, out1, zbuf, stats):
    i = pl.program_id(0)

    @pl.when(i == 0)
    def _():
        stats[...] = jnp.zeros_like(stats)

    @pl.when(i < NB)
    def _():
        _layer_compute(a0, a1, Wa, ba, Wb, bb, zbuf, stats, i)

    @pl.when(i >= NB)
    def _():
        scale, shift = _bn_coeffs(stats, g, b)
        zn = zbuf[pl.ds((i - NB) * BN_, BN_), :] * scale + shift
        out0[...] = zn[:, :HN]
        out1[...] = zn[:, HN:]


def _mlp_pool_head_body(a0, a1, batch, Wa, ba, Wb, bb, g, b,
                        W1, b1, W2, b2, out, stats, pooled):
    i = pl.program_id(0)

    @pl.when(i == 0)
    def _():
        stats[...] = jnp.zeros_like(stats)
        pooled[...] = jnp.zeros_like(pooled)

    u = jnp.concatenate([a0[...], a1[...]], axis=1)
    z = jnp.maximum(_mm(u, Wa[...]) + ba[...], 0.0)
    z = jnp.maximum(_mm(z, Wb[...]) + bb[...], 0.0)
    stats[0:1, :] += jnp.sum(z, axis=0, keepdims=True)
    stats[1:2, :] += jnp.sum(z * z, axis=0, keepdims=True)
    oh = (batch[...] == lax.broadcasted_iota(jnp.int32, (BN_, G), 1))
    oh = oh.astype(jnp.float32)
    zaug = jnp.concatenate(
        [z, jnp.ones((BN_, HN), jnp.float32)], axis=1)
    pooled[...] += lax.dot_general(oh, zaug, (((0,), (0,)), ((), ())),
                                   preferred_element_type=jnp.float32)

    @pl.when(i == NB - 1)
    def _():
        scale, shift = _bn_coeffs(stats, g, b)
        P = pooled[...]
        cnt = P[:, H:H + 1]
        pm = (P[:, :H] / jnp.maximum(cnt, 1.0)) * scale + shift
        o = jnp.maximum(_mm(pm, W1[...]) + b1[...], 0.0)
        o = _mm(o, W2[...]) + b2[...]
        m = jnp.max(o, axis=1, keepdims=True)
        lse = jnp.log(jnp.sum(jnp.exp(o - m), axis=1, keepdims=True))
        out[...] = o - m - lse


def _clampi(i):
    return jnp.minimum(i, NB - 1)


_COMMON_IN_SPECS = [
    pl.BlockSpec((BN_, HN), lambda i: (_clampi(i), 0)),
    pl.BlockSpec((BN_, HN), lambda i: (NB + _clampi(i), 0)),
    pl.BlockSpec((H, H), lambda i: (0, 0)),
    pl.BlockSpec((1, H), lambda i: (0, 0)),
    pl.BlockSpec((H, H), lambda i: (0, 0)),
    pl.BlockSpec((1, H), lambda i: (0, 0)),
    pl.BlockSpec((1, H), lambda i: (0, 0)),
    pl.BlockSpec((1, H), lambda i: (0, 0)),
]

_mlp_mid = pl.pallas_call(
    _mlp_mid_body,
    grid=(2 * NB,),
    in_specs=_COMMON_IN_SPECS,
    out_specs=[
        pl.BlockSpec((BN_, HN), lambda i: (jnp.maximum(i - NB, 0), 0)),
        pl.BlockSpec((BN_, HN), lambda i: (jnp.maximum(i - NB, 0), 0)),
    ],
    out_shape=[jax.ShapeDtypeStruct((N, HN), jnp.float32),
               jax.ShapeDtypeStruct((N, HN), jnp.float32)],
    scratch_shapes=[pltpu.VMEM((N, H), jnp.float32),
                    pltpu.VMEM((8, H), jnp.float32)],
    compiler_params=pltpu.CompilerParams(
        dimension_semantics=("arbitrary",)),
)

_mlp_pool_head = pl.pallas_call(
    _mlp_pool_head_body,
    grid=(NB,),
    in_specs=[
        pl.BlockSpec((BN_, HN), lambda i: (i, 0)),
        pl.BlockSpec((BN_, HN), lambda i: (NB + i, 0)),
        pl.BlockSpec((BN_, 1), lambda i: (i, 0)),
        pl.BlockSpec((H, H), lambda i: (0, 0)),
        pl.BlockSpec((1, H), lambda i: (0, 0)),
        pl.BlockSpec((H, H), lambda i: (0, 0)),
        pl.BlockSpec((1, H), lambda i: (0, 0)),
        pl.BlockSpec((1, H), lambda i: (0, 0)),
        pl.BlockSpec((1, H), lambda i: (0, 0)),
        pl.BlockSpec((H, H), lambda i: (0, 0)),
        pl.BlockSpec((1, H), lambda i: (0, 0)),
        pl.BlockSpec((H, C), lambda i: (0, 0)),
        pl.BlockSpec((1, C), lambda i: (0, 0)),
    ],
    out_specs=pl.BlockSpec((G, C), lambda i: (0, 0)),
    out_shape=jax.ShapeDtypeStruct((G, C), jnp.float32),
    scratch_shapes=[pltpu.VMEM((8, H), jnp.float32),
                    pltpu.VMEM((G, H + HN), jnp.float32)],
    compiler_params=pltpu.CompilerParams(
        dimension_semantics=("arbitrary",)),
)


def kernel(x, edge_index, batch, Wa, ba, Wb, bb, gamma, beta, W1, b1, W2, b2):
    src = edge_index[0].astype(jnp.int32)
    dst = edge_index[1].astype(jnp.int32)
    dst2 = dst
    batch2 = batch.reshape(N, 1).astype(jnp.int32)
    h2a = x[:, :HN]
    h2b = x[:, HN:]
    seg_sum = _get_seg_sum()
    for l in range(NL - 1):
        agg2 = seg_sum(h2a, h2b, src, dst2)
        h2a, h2b = _mlp_mid(agg2, agg2, Wa[l], ba[l].reshape(1, H), Wb[l],
                            bb[l].reshape(1, H), gamma[l].reshape(1, H),
                            beta[l].reshape(1, H))
    agg2 = seg_sum(h2a, h2b, src, dst2)
    l = NL - 1
    return _mlp_pool_head(agg2, agg2, batch2, Wa[l], ba[l].reshape(1, H),
                          Wb[l], bb[l].reshape(1, H),
                          gamma[l].reshape(1, H), beta[l].reshape(1, H),
                          W1, b1.reshape(1, H), W2, b2.reshape(1, C))


# speedup vs baseline: 1.6574x; 1.0102x over previous
import jax
import jax.numpy as jnp
from jax import lax
from jax.experimental import pallas as pl
from jax.experimental.pallas import tpu as pltpu
from jax.experimental.pallas import tpu_sc as plsc

N = 10000
E = 160000
H = 256
NL = 5
G = 64
C = 10

NC = 2
NS = 16
HN = H // 2
EPT = E // NS
K = 80
NCHUNK = EPT // K
RPT = 624
RPT_LAST = N - (NS - 1) * RPT

BN_ = 5000
NB = N // BN_


NSI = 8
NRW = 4


def _seg_sum_body(h2a, h2b, src2, dst2, out,
                  si0, si1, si2, si3, si4, si5, si6, si7,
                  dd0, dd1, dd2, dd3,
                  rw0, rw1, rw2, rw3, acc,
                  is0, is1, is2, is3, is4, is5, is6, is7,
                  dsm0, dsm1, dsm2, dsm3,
                  gs0, gs1, gs2, gs3, ss0, ss1, ss2, ss3):
    sis = [si0, si1, si2, si3, si4, si5, si6, si7]
    dds = [dd0, dd1, dd2, dd3]
    rws = [rw0, rw1, rw2, rw3]
    isem = [is0, is1, is2, is3, is4, is5, is6, is7]
    dsem = [dsm0, dsm1, dsm2, dsm3]
    gsem = [gs0, gs1, gs2, gs3]
    ssem = [ss0, ss1, ss2, ss3]
    c = lax.axis_index("c")
    s = lax.axis_index("s")
    ebase = s * EPT
    dbase = s * EPT

    def iload(j, q):
        pltpu.async_copy(src2.at[pl.ds(ebase + j * K, K)], sis[q], isem[q])

    def dload(j, b):
        pltpu.async_copy(dst2.at[pl.ds(dbase + j * K, K)], dds[b], dsem[b])

    def gather(q, b):
        @pl.when(c == 0)
        def _():
            pltpu.async_copy(h2a.at[sis[q]], rws[b], gsem[b])

        @pl.when(c == 1)
        def _():
            pltpu.async_copy(h2b.at[sis[q]], rws[b], gsem[b])

    def wait_iload(q):
        pltpu.make_async_copy(src2.at[pl.ds(dbase, K)], sis[q],
                              isem[q]).wait()

    def wait_dload(b):
        pltpu.make_async_copy(dst2.at[pl.ds(dbase, K)], dds[b],
                              dsem[b]).wait()

    def wait_gather(b):
        pltpu.make_async_copy(h2a.at[sis[b]], rws[b], gsem[b]).wait()

    def wait_scatter(b):
        pltpu.make_async_copy(rws[b], acc.at[dds[b]], ssem[b]).wait()

    for q in range(NSI):
        iload(q, q)
    for b in range(NRW):
        dload(b, b)

    @pl.when((c == 0) & (s < NS - 1))
    def _():
        pltpu.sync_copy(h2a.at[pl.ds(s * RPT, RPT)],
                        acc.at[pl.ds(s * RPT, RPT)])

    @pl.when((c == 0) & (s == NS - 1))
    def _():
        pltpu.sync_copy(h2a.at[pl.ds((NS - 1) * RPT, RPT_LAST)],
                        acc.at[pl.ds((NS - 1) * RPT, RPT_LAST)])

    @pl.when((c == 1) & (s < NS - 1))
    def _():
        pltpu.sync_copy(h2b.at[pl.ds(s * RPT, RPT)],
                        acc.at[pl.ds(s * RPT, RPT)])

    @pl.when((c == 1) & (s == NS - 1))
    def _():
        pltpu.sync_copy(h2b.at[pl.ds((NS - 1) * RPT, RPT_LAST)],
                        acc.at[pl.ds((NS - 1) * RPT, RPT_LAST)])

    for b in range(NRW):
        wait_iload(b)
        gather(b, b)

    plsc.subcore_barrier()

    GRP = NCHUNK // NSI
    TAIL = GRP * NSI

    def emit(j, u, in_loop):
        b, q = u % NRW, u
        wait_gather(b)
        wait_dload(b)
        pltpu.async_copy(rws[b], acc.at[dds[b]], ssem[b], add=True)
        nxt_load = j + NSI < NCHUNK
        nxt_gather = j + NRW < NCHUNK

        def advance():
            wait_scatter(b)
            dload(j + NRW, b)
            wait_iload((u + NRW) % NSI)
            gather((u + NRW) % NSI, b)

        if in_loop:
            @pl.when(nxt_load)
            def _():
                iload(j + NSI, q)

            @pl.when(nxt_gather)
            def _():
                advance()
        else:
            if nxt_load:
                iload(j + NSI, q)
            if nxt_gather:
                advance()

    def body(g, carry):
        for u in range(NSI):
            emit(NSI * g + u, u, True)
        return carry

    lax.fori_loop(0, GRP, body, 0)
    for j in range(TAIL, NCHUNK):
        emit(j, j % NSI, False)
    for b in range(NRW):
        wait_scatter(b)

    plsc.subcore_barrier()

    @pl.when(s < NS - 1)
    def _():
        pltpu.sync_copy(acc.at[pl.ds(s * RPT, RPT)],
                        out.at[pl.ds(c * N + s * RPT, RPT)])

    @pl.when(s == NS - 1)
    def _():
        pltpu.sync_copy(acc.at[pl.ds((NS - 1) * RPT, RPT_LAST)],
                        out.at[pl.ds(c * N + (NS - 1) * RPT, RPT_LAST)])


_seg_sum_cache = None


def _get_seg_sum():
    global _seg_sum_cache
    if _seg_sum_cache is None:
        _seg_sum_cache = pl.kernel(
            _seg_sum_body,
            out_type=jax.ShapeDtypeStruct((2 * N, HN), jnp.float32),
            mesh=plsc.VectorSubcoreMesh(core_axis_name="c",
                                        subcore_axis_name="s",
                                        num_cores=NC, num_subcores=NS),
            scratch_types=(
                [pltpu.VMEM((K,), jnp.int32) for _ in range(NSI)]
                + [pltpu.VMEM((K,), jnp.int32) for _ in range(NRW)]
                + [pltpu.VMEM((K, HN), jnp.float32) for _ in range(NRW)]
                + [pltpu.VMEM_SHARED((N, HN), jnp.float32)]
                + [pltpu.SemaphoreType.DMA
                   for _ in range(NSI + 3 * NRW)]
            ),
        )
    return _seg_sum_cache


def _mm(a, b):
    return lax.dot_general(a, b, (((1,), (0,)), ((), ())),
                           preferred_element_type=jnp.float32)


def _layer_compute(a0, a1, Wa, ba, Wb, bb, zbuf, stats, i):
    u = jnp.concatenate([a0[...], a1[...]], axis=1)
    z = jnp.maximum(_mm(u, Wa[...]) + ba[...], 0.0)
    z = jnp.maximum(_mm(z, Wb[...]) + bb[...], 0.0)
    zbuf[pl.ds(i * BN_, BN_), :] = z
    stats[0:1, :] += jnp.sum(z, axis=0, keepdims=True)
    stats[1:2, :] += jnp.sum(z * z, axis=0, keepdims=True)


def _bn_coeffs(stats, g, b):
    mu = stats[0:1, :] * (1.0 / N)
    var = stats[1:2, :] * (1.0 / N) - mu * mu
    scale = g[...] * lax.rsqrt(var + 1e-5)
    shift = b[...] - mu * scale
    return scale, shift


def _mlp_mid_body(a0, a1, Wa, ba, Wb, bb, g, b, out0, out1, zbuf, stats):
    i = pl.program_id(0)

    @pl.when(i == 0)
    def _():
        stats[...] = jnp.zeros_like(stats)

    @pl.when(i < NB)
    def _():
        _layer_compute(a0, a1, Wa, ba, Wb, bb, zbuf, stats, i)

    @pl.when(i >= NB)
    def _():
        scale, shift = _bn_coeffs(stats, g, b)
        zn = zbuf[pl.ds((i - NB) * BN_, BN_), :] * scale + shift
        out0[...] = zn[:, :HN]
        out1[...] = zn[:, HN:]


def _mlp_pool_head_body(a0, a1, batch, Wa, ba, Wb, bb, g, b,
                        W1, b1, W2, b2, out, stats, pooled):
    i = pl.program_id(0)

    @pl.when(i == 0)
    def _():
        stats[...] = jnp.zeros_like(stats)
        pooled[...] = jnp.zeros_like(pooled)

    u = jnp.concatenate([a0[...], a1[...]], axis=1)
    z = jnp.maximum(_mm(u, Wa[...]) + ba[...], 0.0)
    z = jnp.maximum(_mm(z, Wb[...]) + bb[...], 0.0)
    stats[0:1, :] += jnp.sum(z, axis=0, keepdims=True)
    stats[1:2, :] += jnp.sum(z * z, axis=0, keepdims=True)
    oh = (batch[...] == lax.broadcasted_iota(jnp.int32, (BN_, G), 1))
    oh = oh.astype(jnp.float32)
    zaug = jnp.concatenate(
        [z, jnp.ones((BN_, HN), jnp.float32)], axis=1)
    pooled[...] += lax.dot_general(oh, zaug, (((0,), (0,)), ((), ())),
                                   preferred_element_type=jnp.float32)

    @pl.when(i == NB - 1)
    def _():
        scale, shift = _bn_coeffs(stats, g, b)
        P = pooled[...]
        cnt = P[:, H:H + 1]
        pm = (P[:, :H] / jnp.maximum(cnt, 1.0)) * scale + shift
        o = jnp.maximum(_mm(pm, W1[...]) + b1[...], 0.0)
        o = _mm(o, W2[...]) + b2[...]
        m = jnp.max(o, axis=1, keepdims=True)
        lse = jnp.log(jnp.sum(jnp.exp(o - m), axis=1, keepdims=True))
        out[...] = o - m - lse


def _clampi(i):
    return jnp.minimum(i, NB - 1)


_COMMON_IN_SPECS = [
    pl.BlockSpec((BN_, HN), lambda i: (_clampi(i), 0)),
    pl.BlockSpec((BN_, HN), lambda i: (NB + _clampi(i), 0)),
    pl.BlockSpec((H, H), lambda i: (0, 0)),
    pl.BlockSpec((1, H), lambda i: (0, 0)),
    pl.BlockSpec((H, H), lambda i: (0, 0)),
    pl.BlockSpec((1, H), lambda i: (0, 0)),
    pl.BlockSpec((1, H), lambda i: (0, 0)),
    pl.BlockSpec((1, H), lambda i: (0, 0)),
]

_mlp_mid = pl.pallas_call(
    _mlp_mid_body,
    grid=(2 * NB,),
    in_specs=_COMMON_IN_SPECS,
    out_specs=[
        pl.BlockSpec((BN_, HN), lambda i: (jnp.maximum(i - NB, 0), 0)),
        pl.BlockSpec((BN_, HN), lambda i: (jnp.maximum(i - NB, 0), 0)),
    ],
    out_shape=[jax.ShapeDtypeStruct((N, HN), jnp.float32),
               jax.ShapeDtypeStruct((N, HN), jnp.float32)],
    scratch_shapes=[pltpu.VMEM((N, H), jnp.float32),
                    pltpu.VMEM((8, H), jnp.float32)],
    compiler_params=pltpu.CompilerParams(
        dimension_semantics=("arbitrary",)),
)

_mlp_pool_head = pl.pallas_call(
    _mlp_pool_head_body,
    grid=(NB,),
    in_specs=[
        pl.BlockSpec((BN_, HN), lambda i: (i, 0)),
        pl.BlockSpec((BN_, HN), lambda i: (NB + i, 0)),
        pl.BlockSpec((BN_, 1), lambda i: (i, 0)),
        pl.BlockSpec((H, H), lambda i: (0, 0)),
        pl.BlockSpec((1, H), lambda i: (0, 0)),
        pl.BlockSpec((H, H), lambda i: (0, 0)),
        pl.BlockSpec((1, H), lambda i: (0, 0)),
        pl.BlockSpec((1, H), lambda i: (0, 0)),
        pl.BlockSpec((1, H), lambda i: (0, 0)),
        pl.BlockSpec((H, H), lambda i: (0, 0)),
        pl.BlockSpec((1, H), lambda i: (0, 0)),
        pl.BlockSpec((H, C), lambda i: (0, 0)),
        pl.BlockSpec((1, C), lambda i: (0, 0)),
    ],
    out_specs=pl.BlockSpec((G, C), lambda i: (0, 0)),
    out_shape=jax.ShapeDtypeStruct((G, C), jnp.float32),
    scratch_shapes=[pltpu.VMEM((8, H), jnp.float32),
                    pltpu.VMEM((G, H + HN), jnp.float32)],
    compiler_params=pltpu.CompilerParams(
        dimension_semantics=("arbitrary",)),
)


def kernel(x, edge_index, batch, Wa, ba, Wb, bb, gamma, beta, W1, b1, W2, b2):
    src = edge_index[0].astype(jnp.int32)
    dst = edge_index[1].astype(jnp.int32)
    dst2 = dst
    batch2 = batch.reshape(N, 1).astype(jnp.int32)
    h2a = x[:, :HN]
    h2b = x[:, HN:]
    seg_sum = _get_seg_sum()
    for l in range(NL - 1):
        agg2 = seg_sum(h2a, h2b, src, dst2)
        h2a, h2b = _mlp_mid(agg2, agg2, Wa[l], ba[l].reshape(1, H), Wb[l],
                            bb[l].reshape(1, H), gamma[l].reshape(1, H),
                            beta[l].reshape(1, H))
    agg2 = seg_sum(h2a, h2b, src, dst2)
    l = NL - 1
    return _mlp_pool_head(agg2, agg2, batch2, Wa[l], ba[l].reshape(1, H),
                          Wb[l], bb[l].reshape(1, H),
                          gamma[l].reshape(1, H), beta[l].reshape(1, H),
                          W1, b1.reshape(1, H), W2, b2.reshape(1, C))
